# Initial kernel scaffold; baseline (speedup 1.0000x reference)
import jax
import jax.numpy as jnp
from jax.experimental import pallas as pl


def kernel(x, edge_index, Wl1, bl1, Wr1, g1, be1, Wl2, bl2, Wr2, g2, be2, Wl3, bl3, Wr3):
    raise NotImplementedError("write your pallas kernel here")


# speedup vs baseline: 4.0739x; 4.0739x over previous
import functools

import jax
import jax.numpy as jnp
from jax import lax
from jax.experimental import pallas as pl
from jax.experimental.pallas import tpu as pltpu
from jax.experimental.pallas import tpu_sc as plsc

NC = 2
NS = 16
CH = 128
D = 128


def _mesh():
    return plsc.VectorSubcoreMesh(core_axis_name="c", subcore_axis_name="s",
                                  num_cores=NC, num_subcores=NS)


def _fill(buf, val):
    v = jnp.full((16,), val, jnp.float32)

    def row(r, _):
        for k in range(D // 16):
            buf[r, pl.ds(16 * k, 16)] = v
        return 0
    lax.fori_loop(0, CH, row, 0)


def _agg_body(n_pad, chunks_per_tile, h_hbm, src_hbm, dst_hbm, aggp,
              idx_s, idx_d, rows, acc, gsem):
    c = lax.axis_index("c")
    s = lax.axis_index("s")
    wid = c * NS + s
    ebase = wid * (chunks_per_tile * CH)
    rows_per_tile = n_pad // NS
    tbase = s * rows_per_tile

    _fill(rows, 0.0)
    for jj in range(rows_per_tile // CH):
        pltpu.sync_copy(rows, acc.at[pl.ds(tbase + jj * CH, CH)])
    plsc.subcore_barrier()

    def body(j, _):
        off = ebase + j * CH
        pltpu.sync_copy(src_hbm.at[pl.ds(off, CH)], idx_s)
        pltpu.async_copy(h_hbm.at[idx_s], rows, gsem).wait()
        pltpu.sync_copy(dst_hbm.at[pl.ds(off, CH)], idx_d)
        pltpu.sync_copy(rows, acc.at[idx_d], add=True)
        return 0
    lax.fori_loop(0, chunks_per_tile, body, 0)
    plsc.subcore_barrier()

    obase = c * n_pad + tbase
    for jj in range(rows_per_tile // CH):
        pltpu.sync_copy(acc.at[pl.ds(tbase + jj * CH, CH)], rows)
        pltpu.sync_copy(rows, aggp.at[pl.ds(obase + jj * CH, CH)])


def _aggregate(h, src_p, dst_p, n_pad):
    e_pad = src_p.shape[0]
    chunks_per_tile = e_pad // (NC * NS * CH)
    f = pl.kernel(
        functools.partial(_agg_body, n_pad, chunks_per_tile),
        out_type=jax.ShapeDtypeStruct((NC * n_pad, D), jnp.float32),
        mesh=_mesh(),
        scratch_types=[
            pltpu.VMEM((CH,), jnp.int32),
            pltpu.VMEM((CH,), jnp.int32),
            pltpu.VMEM((CH, D), jnp.float32),
            pltpu.VMEM_SHARED((n_pad, D), jnp.float32),
            pltpu.SemaphoreType.DMA,
        ],
    )
    return f(h, src_p, dst_p).reshape(NC, n_pad, D)


def _cnt_body(n_pad, chunks_per_tile, dst_hbm, cntp, idx_d, ones, acc, gsem):
    c = lax.axis_index("c")
    s = lax.axis_index("s")
    wid = c * NS + s
    ebase = wid * (chunks_per_tile * CH)
    rows_per_tile = n_pad // NS
    tbase = s * rows_per_tile

    _fill(ones, 0.0)
    for jj in range(rows_per_tile // CH):
        pltpu.sync_copy(ones, acc.at[pl.ds(tbase + jj * CH, CH)])
    _fill(ones, 1.0)
    plsc.subcore_barrier()

    def body(j, _):
        off = ebase + j * CH
        pltpu.sync_copy(dst_hbm.at[pl.ds(off, CH)], idx_d)
        pltpu.sync_copy(ones, acc.at[idx_d], add=True)
        return 0
    lax.fori_loop(0, chunks_per_tile, body, 0)
    plsc.subcore_barrier()

    obase = c * n_pad + tbase
    for jj in range(rows_per_tile // CH):
        pltpu.sync_copy(acc.at[pl.ds(tbase + jj * CH, CH)], ones)
        pltpu.sync_copy(ones, cntp.at[pl.ds(obase + jj * CH, CH)])


def _count(dst_p, n_pad):
    e_pad = dst_p.shape[0]
    chunks_per_tile = e_pad // (NC * NS * CH)
    f = pl.kernel(
        functools.partial(_cnt_body, n_pad, chunks_per_tile),
        out_type=jax.ShapeDtypeStruct((NC * n_pad, D), jnp.float32),
        mesh=_mesh(),
        scratch_types=[
            pltpu.VMEM((CH,), jnp.int32),
            pltpu.VMEM((CH, D), jnp.float32),
            pltpu.VMEM_SHARED((n_pad, D), jnp.float32),
            pltpu.SemaphoreType.DMA,
        ],
    )
    return f(dst_p).reshape(NC, n_pad, D)[:, :, 0]


def _dense_body(relu, aggp, cntp, h, wl, wr, b, o):
    cnt = jnp.sum(cntp[...], axis=0)[:, None]
    inv = 1.0 / jnp.maximum(cnt, 1.0)
    agg = (aggp[0] + aggp[1]) * inv
    acc = jnp.dot(agg, wl[...], preferred_element_type=jnp.float32)
    acc = acc + jnp.dot(h[...], wr[...], preferred_element_type=jnp.float32)
    acc = acc + b[...]
    o[...] = jnp.maximum(acc, 0.0) if relu else acc


def _block(bn, d):
    return pl.BlockSpec((bn, d), lambda i: (i, 0))


def _full(shape):
    return pl.BlockSpec(shape, lambda i: tuple(0 for _ in shape))


def _dense(aggp, cnt2, h, wlT, wrT, b, relu):
    n, d_in = h.shape
    d_out = wlT.shape[1]
    bn = 1024
    return pl.pallas_call(
        functools.partial(_dense_body, relu),
        grid=(pl.cdiv(n, bn),),
        in_specs=[pl.BlockSpec((NC, bn, d_in), lambda i: (0, i, 0)),
                  pl.BlockSpec((NC, bn), lambda i: (0, i)),
                  _block(bn, d_in),
                  _full(wlT.shape), _full(wrT.shape), _full(b.shape)],
        out_specs=_block(bn, d_out),
        out_shape=jax.ShapeDtypeStruct((n, d_out), jnp.float32),
    )(aggp, cnt2, h, wlT, wrT, b)


def kernel(x, edge_index, Wl1, bl1, Wr1, g1, be1,
           Wl2, bl2, Wr2, g2, be2, Wl3, bl3, Wr3):
    n = x.shape[0]
    e = edge_index.shape[1]
    n_pad = ((n + NS * CH - 1) // (NS * CH)) * (NS * CH)
    e_pad = ((e + NC * NS * CH - 1) // (NC * NS * CH)) * (NC * NS * CH)

    src = edge_index[0]
    dst = edge_index[1]
    if e_pad != e:
        pad = e_pad - e
        src = jnp.concatenate([src, jnp.zeros((pad,), jnp.int32)])
        dst = jnp.concatenate([dst, jnp.full((pad,), n_pad - CH, jnp.int32)])

    s1 = g1 / jnp.sqrt(1.0 + 1e-5)
    s2 = g2 / jnp.sqrt(1.0 + 1e-5)
    wl1T = (Wl1 * s1[:, None]).T
    wr1T = (Wr1 * s1[:, None]).T
    b1 = (bl1 * s1 + be1)[None, :]
    wl2T = (Wl2 * s2[:, None]).T
    wr2T = (Wr2 * s2[:, None]).T
    b2 = (bl2 * s2 + be2)[None, :]
    wl3T = Wl3.T
    wr3T = Wr3.T
    b3 = bl3[None, :]

    cnt2 = _count(dst, n_pad)
    aggp1 = _aggregate(x, src, dst, n_pad)
    h1 = _dense(aggp1, cnt2, x, wl1T, wr1T, b1, relu=True)
    aggp2 = _aggregate(h1, src, dst, n_pad)
    h2 = _dense(aggp2, cnt2, h1, wl2T, wr2T, b2, relu=True)
    aggp3 = _aggregate(h2, src, dst, n_pad)
    return _dense(aggp3, cnt2, h2, wl3T, wr3T, b3, relu=False)
